# Initial kernel scaffold; baseline (speedup 1.0000x reference)
import jax
import jax.numpy as jnp
from jax.experimental import pallas as pl


def kernel(x, edge_index, edge_attr, We1, be1, eps1, W1a, b1a, g1a, bt1a, W1b, b1b, gn1, bn1, We2, be2, eps2, W2a, b2a, g2a, bt2a, W2b, b2b, gn2, bn2, Wout, bout):
    raise NotImplementedError("write your pallas kernel here")


# speedup vs baseline: 2.0870x; 2.0870x over previous
import functools

import jax
import jax.numpy as jnp
from jax import lax
from jax.experimental import pallas as pl
from jax.experimental.pallas import tpu as pltpu
from jax.experimental.pallas import tpu_sc as plsc

_N = 10000
_E = 320000
_NC = 2
_NS = 16
_C = 80
_DH = 128


def _edge_proj_body(ea_ref, wt_ref, b_ref, out_ref):
    r = jnp.dot(ea_ref[...], wt_ref[...], preferred_element_type=jnp.float32)
    out_ref[...] = (r + b_ref[...])[None]


def _edge_proj(edge_attr, WeT, be, n_halves, be_blk=4000):
    e, de = edge_attr.shape
    out = pl.pallas_call(
        _edge_proj_body,
        grid=(n_halves, e // be_blk),
        in_specs=[
            pl.BlockSpec((be_blk, de), lambda h, i: (i, 0)),
            pl.BlockSpec((de, _DH), lambda h, i: (0, h)),
            pl.BlockSpec((1, _DH), lambda h, i: (0, h)),
        ],
        out_specs=pl.BlockSpec((1, be_blk, _DH), lambda h, i: (h, i, 0)),
        out_shape=jax.ShapeDtypeStruct((n_halves, e, _DH), jnp.float32),
    )(edge_attr, WeT, be)
    return out.reshape(n_halves * e, _DH)


def _make_sc_aggr(col_split):
    ept = _E // _NS if col_split else _E // (_NC * _NS)
    n_chunks = ept // _C
    rpt = 1000
    n_striped = _N // rpt

    def body(table_hbm, e_hbm, src_hbm, dst_hbm, zero_hbm, out_hbm,
             acc, srcv, dstv, gx, ebuf, sem):
        c = lax.axis_index("c")
        s = lax.axis_index("s")
        r0 = s * rpt

        @pl.when(s < n_striped)
        def _zero():
            pltpu.sync_copy(zero_hbm.at[pl.ds(r0, rpt)],
                            acc.at[pl.ds(r0, rpt)])

        plsc.subcore_barrier()
        tile_base = (s * ept) if col_split else ((c * _NS + s) * ept)

        def chunk(k, carry):
            base = tile_base + k * _C
            ibase = c * _E + base if col_split else base
            pltpu.sync_copy(src_hbm.at[pl.ds(ibase, _C)], srcv)
            pltpu.sync_copy(dst_hbm.at[pl.ds(base, _C)], dstv)
            pltpu.async_copy(table_hbm.at[srcv], gx, sem).wait()
            ebase = ibase
            pltpu.sync_copy(e_hbm.at[pl.ds(ebase, _C)], ebuf)

            def row(i, cr):
                for j in range(_DH // 16):
                    sl = pl.ds(j * 16, 16)
                    gx[i, sl] = jnp.maximum(gx[i, sl] + ebuf[i, sl], 0.0)
                return cr

            lax.fori_loop(0, _C, row, 0)
            pltpu.sync_copy(gx, acc.at[dstv], add=True)
            return carry

        lax.fori_loop(0, n_chunks, chunk, 0)
        plsc.subcore_barrier()

        @pl.when(s < n_striped)
        def _flush():
            pltpu.sync_copy(acc.at[pl.ds(r0, rpt)],
                            out_hbm.at[pl.ds(c * _N + r0, rpt)])

    return pl.kernel(
        body,
        out_type=jax.ShapeDtypeStruct((_NC * _N, _DH), jnp.float32),
        mesh=plsc.VectorSubcoreMesh(core_axis_name="c", subcore_axis_name="s",
                                    num_cores=_NC, num_subcores=_NS),
        scratch_types=[
            pltpu.VMEM_SHARED((_N, _DH), jnp.float32),
            pltpu.VMEM((_C,), jnp.int32),
            pltpu.VMEM((_C,), jnp.int32),
            pltpu.VMEM((_C, _DH), jnp.float32),
            pltpu.VMEM((_C, _DH), jnp.float32),
            pltpu.SemaphoreType.DMA,
        ],
    )


_make_sc_aggr = functools.lru_cache(maxsize=None)(_make_sc_aggr)


def _sc_aggr_call(col_split, table, e, src, dst, zero):
    return _make_sc_aggr(col_split)(table, e, src, dst, zero)


def _bn(h, g, b):
    mu = jnp.mean(h, axis=0, keepdims=True)
    var = jnp.mean((h - mu) ** 2, axis=0, keepdims=True)
    return g * (h - mu) / jnp.sqrt(var + 1e-5) + b


def _node1_body(x_ref, p_ref, eps_ref, wa_ref, ba_ref, g_ref, bt_ref,
                wb_ref, bb_ref, gn_ref, bnb_ref, out_ref):
    z = x_ref[...] * (1.0 + eps_ref[...]) + p_ref[0] + p_ref[1]
    h = jnp.dot(z, wa_ref[...], preferred_element_type=jnp.float32) + ba_ref[...]
    h = jnp.maximum(_bn(h, g_ref[...], bt_ref[...]), 0.0)
    h = jnp.dot(h, wb_ref[...], preferred_element_type=jnp.float32) + bb_ref[...]
    y = jnp.maximum(_bn(h, gn_ref[...], bnb_ref[...]), 0.0)
    out_ref[0] = y[:, :_DH]
    out_ref[1] = y[:, _DH:]


def _node2_body(hs_ref, as_ref, eps_ref, wa_ref, ba_ref, g_ref, bt_ref,
                wb_ref, bb_ref, gn_ref, bnb_ref, wo_ref, bo_ref, out_ref):
    h1 = jnp.concatenate([hs_ref[0], hs_ref[1]], axis=1)
    a2 = jnp.concatenate([as_ref[0], as_ref[1]], axis=1)
    z = h1 * (1.0 + eps_ref[...]) + a2
    h = jnp.dot(z, wa_ref[...], preferred_element_type=jnp.float32) + ba_ref[...]
    h = jnp.maximum(_bn(h, g_ref[...], bt_ref[...]), 0.0)
    h = jnp.dot(h, wb_ref[...], preferred_element_type=jnp.float32) + bb_ref[...]
    h = jnp.maximum(_bn(h, gn_ref[...], bnb_ref[...]), 0.0)
    r = jnp.dot(h, wo_ref[...], preferred_element_type=jnp.float32)
    out_ref[...] = r[:, :1] + bo_ref[...]


def _row(v):
    return v.reshape(1, -1)


def kernel(x, edge_index, edge_attr, We1, be1, eps1, W1a, b1a, g1a, bt1a,
           W1b, b1b, gn1, bn1, We2, be2, eps2, W2a, b2a, g2a, bt2a,
           W2b, b2b, gn2, bn2, Wout, bout):
    f32 = jnp.float32
    src = edge_index[0]
    dst = edge_index[1]
    zero = jnp.zeros((_N, _DH), f32)
    eps1r = eps1.reshape(1, 1)
    eps2r = eps2.reshape(1, 1)

    e1 = _edge_proj(edge_attr, We1.T, _row(be1), 1)
    p1 = _sc_aggr_call(False, x, e1, src, dst, zero)
    h1s = pl.pallas_call(
        _node1_body,
        out_shape=jax.ShapeDtypeStruct((2, _N, _DH), f32),
    )(x, p1.reshape(2, _N, _DH), eps1r, W1a.T, _row(b1a), _row(g1a),
      _row(bt1a), W1b.T, _row(b1b), _row(gn1), _row(bn1))

    e2 = _edge_proj(edge_attr, We2.T, _row(be2), 2)
    src2 = jnp.concatenate([src, src + _N])
    a2 = _sc_aggr_call(True, h1s.reshape(2 * _N, _DH), e2, src2, dst, zero)
    wo = jnp.zeros((_DH, _DH), f32).at[:, 0].set(Wout[0])
    out = pl.pallas_call(
        _node2_body,
        out_shape=jax.ShapeDtypeStruct((_N, 1), f32),
    )(h1s, a2.reshape(2, _N, _DH), eps2r, W2a.T, _row(b2a), _row(g2a),
      _row(bt2a), W2b.T, _row(b2b), _row(gn2), _row(bn2), wo,
      bout.reshape(1, 1))
    return out


# speedup vs baseline: 3.8738x; 1.8561x over previous
import functools

import jax
import jax.numpy as jnp
from jax import lax
from jax.experimental import pallas as pl
from jax.experimental.pallas import tpu as pltpu
from jax.experimental.pallas import tpu_sc as plsc

_N = 10000
_E = 320000
_NC = 2
_NS = 16
_C = 80
_DH = 128


def _edge_proj_body(ea_ref, wt_ref, b_ref, out_ref):
    r = jnp.dot(ea_ref[...], wt_ref[...], preferred_element_type=jnp.float32)
    out_ref[...] = (r + b_ref[...])[None]


def _edge_proj(edge_attr, WeT, be, n_halves, be_blk=4000):
    e, de = edge_attr.shape
    out = pl.pallas_call(
        _edge_proj_body,
        grid=(n_halves, e // be_blk),
        in_specs=[
            pl.BlockSpec((be_blk, de), lambda h, i: (i, 0)),
            pl.BlockSpec((de, _DH), lambda h, i: (0, h)),
            pl.BlockSpec((1, _DH), lambda h, i: (0, h)),
        ],
        out_specs=pl.BlockSpec((1, be_blk, _DH), lambda h, i: (h, i, 0)),
        out_shape=jax.ShapeDtypeStruct((n_halves, e, _DH), jnp.float32),
    )(edge_attr, WeT, be)
    return out.reshape(n_halves * e, _DH)


def _make_sc_aggr(col_split):
    ept = _E // _NS if col_split else _E // (_NC * _NS)
    n_chunks = ept // _C
    rpt = 1000
    n_striped = _N // rpt

    def body(table_hbm, e_hbm, src_hbm, dst_hbm, zero_hbm, out_hbm,
             acc, sv0, sv1, dv0, dv1, gx0, gx1, eb0, eb1,
             is0, is1, gs0, gs1, es0, es1):
        sv = (sv0, sv1)
        dv = (dv0, dv1)
        gx = (gx0, gx1)
        eb = (eb0, eb1)
        isem = (is0, is1)
        gsem = (gs0, gs1)
        esem = (es0, es1)
        c = lax.axis_index("c")
        s = lax.axis_index("s")
        r0 = s * rpt

        @pl.when(s < n_striped)
        def _zero():
            pltpu.sync_copy(zero_hbm.at[pl.ds(r0, rpt)],
                            acc.at[pl.ds(r0, rpt)])

        plsc.subcore_barrier()
        tile_base = (s * ept) if col_split else ((c * _NS + s) * ept)
        ibase0 = (c * _E + tile_base) if col_split else tile_base

        def issue_idx(k, b):
            pltpu.async_copy(src_hbm.at[pl.ds(ibase0 + k * _C, _C)],
                             sv[b], isem[b])
            pltpu.async_copy(dst_hbm.at[pl.ds(tile_base + k * _C, _C)],
                             dv[b], isem[b])

        def wait_idx(k, b):
            pltpu.make_async_copy(src_hbm.at[pl.ds(ibase0, _C)],
                                  sv[b], isem[b]).wait()
            pltpu.make_async_copy(dst_hbm.at[pl.ds(tile_base, _C)],
                                  dv[b], isem[b]).wait()

        def issue_data(k, b):
            pltpu.async_copy(table_hbm.at[sv[b]], gx[b], gsem[b])
            pltpu.async_copy(e_hbm.at[pl.ds(ibase0 + k * _C, _C)],
                             eb[b], esem[b])

        def step(k, b, first, last):
            if not last:
                def _launch_next():
                    wait_idx(k + 1, 1 - b)
                    issue_data(k + 1, 1 - b)
                if first:
                    _launch_next()
                else:
                    pl.when(k + 1 < n_chunks)(_launch_next)
            pltpu.make_async_copy(table_hbm.at[sv[b]], gx[b],
                                  gsem[b]).wait()
            pltpu.make_async_copy(e_hbm.at[pl.ds(ibase0, _C)],
                                  eb[b], esem[b]).wait()

            def row(i, cr):
                for j in range(_DH // 16):
                    sl = pl.ds(j * 16, 16)
                    gx[b][i, sl] = jnp.maximum(gx[b][i, sl] + eb[b][i, sl],
                                               0.0)
                return cr

            lax.fori_loop(0, _C, row, 0)
            pltpu.sync_copy(gx[b], acc.at[dv[b]], add=True)
            if not last:
                @pl.when(k + 2 < n_chunks)
                def _refill_idx():
                    issue_idx(k + 2, b)

        issue_idx(0, 0)
        issue_idx(1, 1)
        wait_idx(0, 0)
        issue_data(0, 0)

        step(0, 0, True, False)
        step(1, 1, False, False)

        def pair(g, carry):
            step(2 * g, 0, False, False)
            step(2 * g + 1, 1, False, False)
            return carry

        lax.fori_loop(1, n_chunks // 2, pair, 0)
        if n_chunks % 2 == 1:
            step(n_chunks - 1, 0, False, True)
        plsc.subcore_barrier()

        @pl.when(s < n_striped)
        def _flush():
            pltpu.sync_copy(acc.at[pl.ds(r0, rpt)],
                            out_hbm.at[pl.ds(c * _N + r0, rpt)])

    return pl.kernel(
        body,
        out_type=jax.ShapeDtypeStruct((_NC * _N, _DH), jnp.float32),
        mesh=plsc.VectorSubcoreMesh(core_axis_name="c", subcore_axis_name="s",
                                    num_cores=_NC, num_subcores=_NS),
        scratch_types=[
            pltpu.VMEM_SHARED((_N, _DH), jnp.float32),
            pltpu.VMEM((_C,), jnp.int32),
            pltpu.VMEM((_C,), jnp.int32),
            pltpu.VMEM((_C,), jnp.int32),
            pltpu.VMEM((_C,), jnp.int32),
            pltpu.VMEM((_C, _DH), jnp.float32),
            pltpu.VMEM((_C, _DH), jnp.float32),
            pltpu.VMEM((_C, _DH), jnp.float32),
            pltpu.VMEM((_C, _DH), jnp.float32),
            pltpu.SemaphoreType.DMA,
            pltpu.SemaphoreType.DMA,
            pltpu.SemaphoreType.DMA,
            pltpu.SemaphoreType.DMA,
            pltpu.SemaphoreType.DMA,
            pltpu.SemaphoreType.DMA,
        ],
    )


_make_sc_aggr = functools.lru_cache(maxsize=None)(_make_sc_aggr)


def _sc_aggr_call(col_split, table, e, src, dst, zero):
    return _make_sc_aggr(col_split)(table, e, src, dst, zero)


def _bn(h, g, b):
    mu = jnp.mean(h, axis=0, keepdims=True)
    var = jnp.mean((h - mu) ** 2, axis=0, keepdims=True)
    return g * (h - mu) / jnp.sqrt(var + 1e-5) + b


def _node1_body(x_ref, p_ref, eps_ref, wa_ref, ba_ref, g_ref, bt_ref,
                wb_ref, bb_ref, gn_ref, bnb_ref, out_ref):
    z = x_ref[...] * (1.0 + eps_ref[...]) + p_ref[0] + p_ref[1]
    h = jnp.dot(z, wa_ref[...], preferred_element_type=jnp.float32) + ba_ref[...]
    h = jnp.maximum(_bn(h, g_ref[...], bt_ref[...]), 0.0)
    h = jnp.dot(h, wb_ref[...], preferred_element_type=jnp.float32) + bb_ref[...]
    y = jnp.maximum(_bn(h, gn_ref[...], bnb_ref[...]), 0.0)
    out_ref[0] = y[:, :_DH]
    out_ref[1] = y[:, _DH:]


def _node2_body(hs_ref, as_ref, eps_ref, wa_ref, ba_ref, g_ref, bt_ref,
                wb_ref, bb_ref, gn_ref, bnb_ref, wo_ref, bo_ref, out_ref):
    h1 = jnp.concatenate([hs_ref[0], hs_ref[1]], axis=1)
    a2 = jnp.concatenate([as_ref[0], as_ref[1]], axis=1)
    z = h1 * (1.0 + eps_ref[...]) + a2
    h = jnp.dot(z, wa_ref[...], preferred_element_type=jnp.float32) + ba_ref[...]
    h = jnp.maximum(_bn(h, g_ref[...], bt_ref[...]), 0.0)
    h = jnp.dot(h, wb_ref[...], preferred_element_type=jnp.float32) + bb_ref[...]
    h = jnp.maximum(_bn(h, gn_ref[...], bnb_ref[...]), 0.0)
    r = jnp.dot(h, wo_ref[...], preferred_element_type=jnp.float32)
    out_ref[...] = r[:, :1] + bo_ref[...]


def _row(v):
    return v.reshape(1, -1)


def kernel(x, edge_index, edge_attr, We1, be1, eps1, W1a, b1a, g1a, bt1a,
           W1b, b1b, gn1, bn1, We2, be2, eps2, W2a, b2a, g2a, bt2a,
           W2b, b2b, gn2, bn2, Wout, bout):
    f32 = jnp.float32
    src = edge_index[0]
    dst = edge_index[1]
    zero = jnp.zeros((_N, _DH), f32)
    eps1r = eps1.reshape(1, 1)
    eps2r = eps2.reshape(1, 1)

    e1 = _edge_proj(edge_attr, We1.T, _row(be1), 1)
    p1 = _sc_aggr_call(False, x, e1, src, dst, zero)
    h1s = pl.pallas_call(
        _node1_body,
        out_shape=jax.ShapeDtypeStruct((2, _N, _DH), f32),
    )(x, p1.reshape(2, _N, _DH), eps1r, W1a.T, _row(b1a), _row(g1a),
      _row(bt1a), W1b.T, _row(b1b), _row(gn1), _row(bn1))

    e2 = _edge_proj(edge_attr, We2.T, _row(be2), 2)
    src2 = jnp.concatenate([src, src + _N])
    a2 = _sc_aggr_call(True, h1s.reshape(2 * _N, _DH), e2, src2, dst, zero)
    wo = jnp.zeros((_DH, _DH), f32).at[:, 0].set(Wout[0])
    out = pl.pallas_call(
        _node2_body,
        out_shape=jax.ShapeDtypeStruct((_N, 1), f32),
    )(h1s, a2.reshape(2, _N, _DH), eps2r, W2a.T, _row(b2a), _row(g2a),
      _row(bt2a), W2b.T, _row(b2b), _row(gn2), _row(bn2), wo,
      bout.reshape(1, 1))
    return out


# speedup vs baseline: 4.2408x; 1.0947x over previous
import functools

import jax
import jax.numpy as jnp
from jax import lax
from jax.experimental import pallas as pl
from jax.experimental.pallas import tpu as pltpu
from jax.experimental.pallas import tpu_sc as plsc

_N = 10000
_E = 320000
_NC = 2
_NS = 16
_C = 80
_DH = 128


def _edge_proj_body(ea_ref, wt_ref, b_ref, out_ref):
    r = jnp.dot(ea_ref[...], wt_ref[...], preferred_element_type=jnp.float32)
    out_ref[...] = (r + b_ref[...])[None]


def _edge_proj(edge_attr, WeT, be, n_halves, be_blk=4000):
    e, de = edge_attr.shape
    out = pl.pallas_call(
        _edge_proj_body,
        grid=(n_halves, e // be_blk),
        in_specs=[
            pl.BlockSpec((be_blk, de), lambda h, i: (i, 0)),
            pl.BlockSpec((de, _DH), lambda h, i: (0, h)),
            pl.BlockSpec((1, _DH), lambda h, i: (0, h)),
        ],
        out_specs=pl.BlockSpec((1, be_blk, _DH), lambda h, i: (h, i, 0)),
        out_shape=jax.ShapeDtypeStruct((n_halves, e, _DH), jnp.float32),
    )(edge_attr, WeT, be)
    return out.reshape(n_halves * e, _DH)


def _make_sc_aggr(col_split):
    ept = _E // _NS if col_split else _E // (_NC * _NS)
    n_chunks = ept // _C
    rpt = 1000
    n_striped = _N // rpt

    def body(table_hbm, e_hbm, src_hbm, dst_hbm, zero_hbm, out_hbm,
             acc, sv0, sv1, dv0, dv1, gx0, gx1, eb0, eb1,
             is0, is1, gs0, gs1, es0, es1):
        sv = (sv0, sv1)
        dv = (dv0, dv1)
        gx = (gx0, gx1)
        eb = (eb0, eb1)
        isem = (is0, is1)
        gsem = (gs0, gs1)
        esem = (es0, es1)
        c = lax.axis_index("c")
        s = lax.axis_index("s")
        r0 = s * rpt

        @pl.when(s < n_striped)
        def _zero():
            pltpu.sync_copy(zero_hbm.at[pl.ds(r0, rpt)],
                            acc.at[pl.ds(r0, rpt)])

        plsc.subcore_barrier()
        tile_base = (s * ept) if col_split else ((c * _NS + s) * ept)
        ibase0 = (c * _E + tile_base) if col_split else tile_base

        def issue_idx(k, b):
            pltpu.async_copy(src_hbm.at[pl.ds(ibase0 + k * _C, _C)],
                             sv[b], isem[b])
            pltpu.async_copy(dst_hbm.at[pl.ds(tile_base + k * _C, _C)],
                             dv[b], isem[b])

        def wait_idx(k, b):
            pltpu.make_async_copy(src_hbm.at[pl.ds(ibase0, _C)],
                                  sv[b], isem[b]).wait()
            pltpu.make_async_copy(dst_hbm.at[pl.ds(tile_base, _C)],
                                  dv[b], isem[b]).wait()

        def issue_data(k, b):
            pltpu.async_copy(table_hbm.at[sv[b]], gx[b], gsem[b])
            pltpu.async_copy(e_hbm.at[pl.ds(ibase0 + k * _C, _C)],
                             eb[b], esem[b])

        def step(k, b, first, last):
            if not last:
                def _launch_next():
                    wait_idx(k + 1, 1 - b)
                    issue_data(k + 1, 1 - b)
                if first:
                    _launch_next()
                else:
                    pl.when(k + 1 < n_chunks)(_launch_next)
            pltpu.make_async_copy(table_hbm.at[sv[b]], gx[b],
                                  gsem[b]).wait()
            pltpu.make_async_copy(e_hbm.at[pl.ds(ibase0, _C)],
                                  eb[b], esem[b]).wait()

            @functools.partial(plsc.parallel_loop, 0, _C, unroll=4)
            def _rows(i):
                for j in range(_DH // 16):
                    sl = pl.ds(j * 16, 16)
                    gx[b][i, sl] = jnp.maximum(gx[b][i, sl] + eb[b][i, sl],
                                               0.0)
            pltpu.sync_copy(gx[b], acc.at[dv[b]], add=True)
            if not last:
                @pl.when(k + 2 < n_chunks)
                def _refill_idx():
                    issue_idx(k + 2, b)

        issue_idx(0, 0)
        issue_idx(1, 1)
        wait_idx(0, 0)
        issue_data(0, 0)

        step(0, 0, True, False)
        step(1, 1, False, False)

        def pair(g, carry):
            step(2 * g, 0, False, False)
            step(2 * g + 1, 1, False, False)
            return carry

        lax.fori_loop(1, n_chunks // 2, pair, 0)
        if n_chunks % 2 == 1:
            step(n_chunks - 1, 0, False, True)
        plsc.subcore_barrier()

        @pl.when(s < n_striped)
        def _flush():
            pltpu.sync_copy(acc.at[pl.ds(r0, rpt)],
                            out_hbm.at[pl.ds(c * _N + r0, rpt)])

    return pl.kernel(
        body,
        out_type=jax.ShapeDtypeStruct((_NC * _N, _DH), jnp.float32),
        mesh=plsc.VectorSubcoreMesh(core_axis_name="c", subcore_axis_name="s",
                                    num_cores=_NC, num_subcores=_NS),
        scratch_types=[
            pltpu.VMEM_SHARED((_N, _DH), jnp.float32),
            pltpu.VMEM((_C,), jnp.int32),
            pltpu.VMEM((_C,), jnp.int32),
            pltpu.VMEM((_C,), jnp.int32),
            pltpu.VMEM((_C,), jnp.int32),
            pltpu.VMEM((_C, _DH), jnp.float32),
            pltpu.VMEM((_C, _DH), jnp.float32),
            pltpu.VMEM((_C, _DH), jnp.float32),
            pltpu.VMEM((_C, _DH), jnp.float32),
            pltpu.SemaphoreType.DMA,
            pltpu.SemaphoreType.DMA,
            pltpu.SemaphoreType.DMA,
            pltpu.SemaphoreType.DMA,
            pltpu.SemaphoreType.DMA,
            pltpu.SemaphoreType.DMA,
        ],
    )


_make_sc_aggr = functools.lru_cache(maxsize=None)(_make_sc_aggr)


def _sc_aggr_call(col_split, table, e, src, dst, zero):
    return _make_sc_aggr(col_split)(table, e, src, dst, zero)


def _bn(h, g, b):
    mu = jnp.mean(h, axis=0, keepdims=True)
    var = jnp.mean((h - mu) ** 2, axis=0, keepdims=True)
    return g * (h - mu) / jnp.sqrt(var + 1e-5) + b


def _node1_body(x_ref, p_ref, eps_ref, wa_ref, ba_ref, g_ref, bt_ref,
                wb_ref, bb_ref, gn_ref, bnb_ref, out_ref):
    z = x_ref[...] * (1.0 + eps_ref[...]) + p_ref[0] + p_ref[1]
    h = jnp.dot(z, wa_ref[...], preferred_element_type=jnp.float32) + ba_ref[...]
    h = jnp.maximum(_bn(h, g_ref[...], bt_ref[...]), 0.0)
    h = jnp.dot(h, wb_ref[...], preferred_element_type=jnp.float32) + bb_ref[...]
    y = jnp.maximum(_bn(h, gn_ref[...], bnb_ref[...]), 0.0)
    out_ref[0] = y[:, :_DH]
    out_ref[1] = y[:, _DH:]


def _node2_body(hs_ref, as_ref, eps_ref, wa_ref, ba_ref, g_ref, bt_ref,
                wb_ref, bb_ref, gn_ref, bnb_ref, wo_ref, bo_ref, out_ref):
    h1 = jnp.concatenate([hs_ref[0], hs_ref[1]], axis=1)
    a2 = jnp.concatenate([as_ref[0], as_ref[1]], axis=1)
    z = h1 * (1.0 + eps_ref[...]) + a2
    h = jnp.dot(z, wa_ref[...], preferred_element_type=jnp.float32) + ba_ref[...]
    h = jnp.maximum(_bn(h, g_ref[...], bt_ref[...]), 0.0)
    h = jnp.dot(h, wb_ref[...], preferred_element_type=jnp.float32) + bb_ref[...]
    h = jnp.maximum(_bn(h, gn_ref[...], bnb_ref[...]), 0.0)
    r = jnp.dot(h, wo_ref[...], preferred_element_type=jnp.float32)
    out_ref[...] = r[:, :1] + bo_ref[...]


def _row(v):
    return v.reshape(1, -1)


def kernel(x, edge_index, edge_attr, We1, be1, eps1, W1a, b1a, g1a, bt1a,
           W1b, b1b, gn1, bn1, We2, be2, eps2, W2a, b2a, g2a, bt2a,
           W2b, b2b, gn2, bn2, Wout, bout):
    f32 = jnp.float32
    src = edge_index[0]
    dst = edge_index[1]
    zero = jnp.zeros((_N, _DH), f32)
    eps1r = eps1.reshape(1, 1)
    eps2r = eps2.reshape(1, 1)

    e1 = _edge_proj(edge_attr, We1.T, _row(be1), 1)
    e2 = _edge_proj(edge_attr, We2.T, _row(be2), 2)
    p1 = _sc_aggr_call(False, x, e1, src, dst, zero)
    h1s = pl.pallas_call(
        _node1_body,
        out_shape=jax.ShapeDtypeStruct((2, _N, _DH), f32),
    )(x, p1.reshape(2, _N, _DH), eps1r, W1a.T, _row(b1a), _row(g1a),
      _row(bt1a), W1b.T, _row(b1b), _row(gn1), _row(bn1))

    src2 = jnp.concatenate([src, src + _N])
    a2 = _sc_aggr_call(True, h1s.reshape(2 * _N, _DH), e2, src2, dst, zero)
    wo = jnp.zeros((_DH, _DH), f32).at[:, 0].set(Wout[0])
    out = pl.pallas_call(
        _node2_body,
        out_shape=jax.ShapeDtypeStruct((_N, 1), f32),
    )(h1s, a2.reshape(2, _N, _DH), eps2r, W2a.T, _row(b2a), _row(g2a),
      _row(bt2a), W2b.T, _row(b2b), _row(gn2), _row(bn2), wo,
      bout.reshape(1, 1))
    return out
